# Initial kernel scaffold; baseline (speedup 1.0000x reference)
import jax
import jax.numpy as jnp
from jax.experimental import pallas as pl


def kernel(x, edge_index, edge_attr, W_l, b_l, W_r, b_r, W_e, att, bias):
    raise NotImplementedError("write your pallas kernel here")


# speedup vs baseline: 2.5753x; 2.5753x over previous
import functools

import jax
import jax.numpy as jnp
from jax import lax
from jax.experimental import pallas as pl
from jax.experimental.pallas import tpu as pltpu
from jax.experimental.pallas import tpu_sc as plsc

N = 10000
E = 320000
D = 128
DE = 4

NC = 2
NS = 16
NW = NC * NS
L = 16
NJ = D // L

C = 80
G = C // L
EPT = E // NW
EPS = E // NS
NPAD = 10240
STRIPE = NPAD // NS

_mesh = plsc.VectorSubcoreMesh(
    core_axis_name="c", subcore_axis_name="s", num_cores=NC, num_subcores=NS)

_f32 = jnp.float32
_i32 = jnp.int32


def _hsum(v, rbuf):
    s = v
    for sh in (8, 4, 2, 1):
        rbuf[pl.ds(0, L)] = s
        s = s + rbuf[pl.ds(sh, L)]
    return s


def _hmax(v, rbuf):
    s = v
    for sh in (8, 4, 2, 1):
        rbuf[pl.ds(0, L)] = s
        s = jnp.maximum(s, rbuf[pl.ds(sh, L)])
    return s


_IOTA = None


def _bcast0(v, rbufp, iota):
    s = v
    for sh in (1, 2, 4, 8):
        rbufp[pl.ds(2 * L, L)] = s
        shifted = rbufp[pl.ds(2 * L - sh, L)]
        take = (iota & (2 * sh - 1)) >= sh
        s = jnp.where(take, shifted, s)
    return s


def _k1_body(x_ref, wl_ref, bl_ref, wr_ref, br_ref, ol_ref, or_ref):
    xb = x_ref[...]
    ol_ref[...] = jnp.dot(xb, wl_ref[...],
                          preferred_element_type=jnp.float32) + bl_ref[...]
    or_ref[...] = jnp.dot(xb, wr_ref[...],
                          preferred_element_type=jnp.float32) + br_ref[...]


def _k1(x, W_l, b_l, W_r, b_r):
    blk = 2000
    return pl.pallas_call(
        _k1_body,
        grid=(N // blk,),
        in_specs=[
            pl.BlockSpec((blk, D), lambda i: (i, 0)),
            pl.BlockSpec((D, D), lambda i: (0, 0)),
            pl.BlockSpec((1, D), lambda i: (0, 0)),
            pl.BlockSpec((D, D), lambda i: (0, 0)),
            pl.BlockSpec((1, D), lambda i: (0, 0)),
        ],
        out_specs=[
            pl.BlockSpec((blk, D), lambda i: (i, 0)),
            pl.BlockSpec((blk, D), lambda i: (i, 0)),
        ],
        out_shape=[
            jax.ShapeDtypeStruct((N, D), _f32),
            jax.ShapeDtypeStruct((N, D), _f32),
        ],
    )(x, W_l, b_l.reshape(1, D), W_r, b_r.reshape(1, D))


@functools.partial(
    pl.kernel,
    out_type=(
        jax.ShapeDtypeStruct((E,), _f32),
        jax.ShapeDtypeStruct((NW, D), _f32),
    ),
    mesh=_mesh,
    scratch_types=[
        pltpu.VMEM((C,), _i32),
        pltpu.VMEM((C,), _i32),
        pltpu.VMEM((DE * C + L,), _f32),
        pltpu.VMEM((C, D), _f32),
        pltpu.VMEM((C, D), _f32),
        pltpu.VMEM((C,), _f32),
        pltpu.VMEM((D,), _f32),
        pltpu.VMEM((DE, D), _f32),
        pltpu.VMEM((L,), _f32),
        pltpu.VMEM((D,), _f32),
        pltpu.VMEM((2 * L,), _f32),
        pltpu.VMEM((3 * L,), _f32),
        pltpu.SemaphoreType.DMA,
        pltpu.SemaphoreType.DMA,
    ],
)
def _k2(xl_hbm, xr_hbm, src_hbm, dst_hbm, eaf_hbm, we_hbm, att_hbm,
        logits_o, tmax_o,
        src_v, dst_v, ea_flat, xl_rows, xr_rows, lbuf, att_vm, we_vm, maxs,
        maxd, rbuf, rbufp, sem1, sem2):
    cid = lax.axis_index("c")
    sid = lax.axis_index("s")
    wid = sid * NC + cid

    pltpu.sync_copy(att_hbm, att_vm)
    pltpu.sync_copy(we_hbm, we_vm)
    maxs[...] = jnp.full((L,), -1e30, _f32)
    iota = lax.iota(_i32, L)

    att_vecs = [att_vm[pl.ds(j * L, L)] for j in range(NJ)]
    we_vecs = [[we_vm[k, pl.ds(j * L, L)] for j in range(NJ)]
               for k in range(DE)]

    def chunk(ci, carry):
        base = wid * EPT + ci * C
        pltpu.sync_copy(src_hbm.at[pl.ds(base, C)], src_v)
        pltpu.sync_copy(dst_hbm.at[pl.ds(base, C)], dst_v)
        pltpu.sync_copy(eaf_hbm.at[pl.ds(base * DE, C * DE)],
                        ea_flat.at[pl.ds(0, C * DE)])
        cp1 = pltpu.async_copy(xl_hbm.at[src_v], xl_rows, sem1)
        cp2 = pltpu.async_copy(xr_hbm.at[dst_v], xr_rows, sem2)
        cp1.wait()
        cp2.wait()

        for g in range(G):
            def eb(i, lvec, g=g):
                e = g * L + i
                eav = ea_flat[pl.ds(e * DE, L)]
                e0, e1, e2, e3 = eav[0], eav[1], eav[2], eav[3]
                acc = jnp.zeros((L,), _f32)
                for j in range(NJ):
                    sl = pl.ds(j * L, L)
                    ev = (e0 * we_vecs[0][j] + e1 * we_vecs[1][j]
                          + e2 * we_vecs[2][j] + e3 * we_vecs[3][j])
                    h = xl_rows[e, sl] + xr_rows[e, sl] + ev
                    lr = jnp.maximum(h, 0.2 * h)
                    acc = acc + att_vecs[j] * lr
                s = _hsum(acc, rbuf)
                sp = _bcast0(s, rbufp, iota)
                return jnp.where(iota == i, sp, lvec)

            lvec = lax.fori_loop(0, L, eb, jnp.zeros((L,), _f32))
            lbuf[pl.ds(g * L, L)] = lvec
            maxs[...] = jnp.maximum(maxs[...], lvec)

        pltpu.sync_copy(lbuf, logits_o.at[pl.ds(base, C)])
        return carry

    lax.fori_loop(0, EPT // C, chunk, 0)
    mfin = maxs[...]
    for j in range(NJ):
        maxd[pl.ds(j * L, L)] = mfin
    pltpu.sync_copy(maxd, tmax_o.at[wid])


@functools.partial(
    pl.kernel,
    out_type=jax.ShapeDtypeStruct((E,), _f32),
    mesh=_mesh,
    scratch_types=[
        pltpu.VMEM((NW, D), _f32),
        pltpu.VMEM((C,), _i32),
        pltpu.VMEM((C + L,), _f32),
        pltpu.VMEM((C, D), _f32),
        pltpu.VMEM((C, D), _f32),
        pltpu.VMEM((C,), _f32),
        pltpu.VMEM((2 * L,), _f32),
        pltpu.VMEM((3 * L,), _f32),
        pltpu.VMEM_SHARED((NPAD, D), _f32),
        pltpu.SemaphoreType.DMA,
    ],
)
def _k3a(dst_hbm, logits_hbm, tmax_hbm, z128_hbm,
         alpha_o,
         tmaxbuf, dstb, lb, exbuf, den_rows, alb, rbuf, rbufp,
         spmem_den, sem1):
    cid = lax.axis_index("c")
    sid = lax.axis_index("s")
    off = sid * STRIPE
    iota = lax.iota(_i32, L)

    pltpu.sync_copy(tmax_hbm, tmaxbuf)
    mv = tmaxbuf[0, pl.ds(0, L)]
    for t in range(1, NW):
        mv = jnp.maximum(mv, tmaxbuf[t, pl.ds(0, L)])
    M = _bcast0(_hmax(mv, rbuf), rbufp, iota)

    pltpu.sync_copy(z128_hbm, spmem_den.at[pl.ds(off, STRIPE), :])
    plsc.subcore_barrier()

    def achunk(ci, carry):
        base = sid * EPS + ci * C
        pltpu.sync_copy(dst_hbm.at[pl.ds(base, C)], dstb)
        pltpu.sync_copy(logits_hbm.at[pl.ds(base, C)],
                        lb.at[pl.ds(0, C)])

        for g in range(G):
            ex16 = jnp.exp(lb[pl.ds(g * L, L)] - M)
            rbufp[pl.ds(0, L)] = ex16

            def eb(i, carry2, g=g):
                sp = _bcast0(rbufp[pl.ds(i, L)], rbufp, iota)
                for j in range(NJ):
                    exbuf[g * L + i, pl.ds(j * L, L)] = sp
                return carry2
            lax.fori_loop(0, L, eb, 0)
        pltpu.sync_copy(exbuf, spmem_den.at[dstb], add=True)
        return carry
    lax.fori_loop(0, EPS // C, achunk, 0)

    plsc.subcore_barrier()

    def alchunk(ci, carry):
        base = cid * (E // NC) + sid * EPT + ci * C
        pltpu.sync_copy(dst_hbm.at[pl.ds(base, C)], dstb)
        pltpu.sync_copy(logits_hbm.at[pl.ds(base, C)],
                        lb.at[pl.ds(0, C)])
        pltpu.async_copy(spmem_den.at[dstb], den_rows, sem1).wait()

        for g in range(G):
            ex16 = jnp.exp(lb[pl.ds(g * L, L)] - M)
            rbufp[pl.ds(0, L)] = ex16

            def eb(i, alvec, g=g):
                e = g * L + i
                sp = _bcast0(rbufp[pl.ds(i, L)], rbufp, iota)
                al = sp / (den_rows[e, pl.ds(0, L)] + 1e-16)
                return jnp.where(iota == i, al, alvec)
            alvec = lax.fori_loop(0, L, eb, jnp.zeros((L,), _f32))
            alb[pl.ds(g * L, L)] = alvec
        pltpu.sync_copy(alb, alpha_o.at[pl.ds(base, C)])
        return carry
    lax.fori_loop(0, EPT // C, alchunk, 0)


@functools.partial(
    pl.kernel,
    out_type=jax.ShapeDtypeStruct((NC, NPAD, D), _f32),
    mesh=_mesh,
    scratch_types=[
        pltpu.VMEM((C,), _i32),
        pltpu.VMEM((C,), _i32),
        pltpu.VMEM((C + L,), _f32),
        pltpu.VMEM((C, D), _f32),
        pltpu.VMEM((C, D), _f32),
        pltpu.VMEM((3 * L,), _f32),
        pltpu.VMEM_SHARED((NPAD, D), _f32),
        pltpu.SemaphoreType.DMA,
    ],
)
def _k3b(xl_hbm, src_hbm, dst_hbm, alpha_hbm, z128_hbm,
         outp_o,
         srcb, dstb, ab, xl_rows, contrib, rbufp, spmem_out, sem1):
    cid = lax.axis_index("c")
    sid = lax.axis_index("s")
    off = sid * STRIPE
    iota = lax.iota(_i32, L)

    pltpu.sync_copy(z128_hbm, spmem_out.at[pl.ds(off, STRIPE), :])
    plsc.subcore_barrier()

    def bchunk(ci, carry):
        base = cid * (E // NC) + sid * EPT + ci * C
        pltpu.sync_copy(src_hbm.at[pl.ds(base, C)], srcb)
        pltpu.sync_copy(dst_hbm.at[pl.ds(base, C)], dstb)
        pltpu.sync_copy(alpha_hbm.at[pl.ds(base, C)],
                        ab.at[pl.ds(0, C)])
        pltpu.async_copy(xl_hbm.at[srcb], xl_rows, sem1).wait()

        for g in range(G):
            rbufp[pl.ds(0, L)] = ab[pl.ds(g * L, L)]

            def eb(i, carry2, g=g):
                e = g * L + i
                sp = _bcast0(rbufp[pl.ds(i, L)], rbufp, iota)
                for j in range(NJ):
                    sl = pl.ds(j * L, L)
                    contrib[e, sl] = sp * xl_rows[e, sl]
                return carry2
            lax.fori_loop(0, L, eb, 0)
        pltpu.sync_copy(contrib, spmem_out.at[dstb], add=True)
        return carry
    lax.fori_loop(0, EPT // C, bchunk, 0)

    plsc.subcore_barrier()
    pltpu.sync_copy(spmem_out.at[pl.ds(off, STRIPE), :],
                    outp_o.at[cid, pl.ds(off, STRIPE), :])


def _k4_body(p_ref, b_ref, o_ref):
    o_ref[...] = jnp.maximum(p_ref[0] + p_ref[1] + b_ref[...], 0.0)


def _k4(outp, bias):
    blk = 1000
    return pl.pallas_call(
        _k4_body,
        grid=(N // blk,),
        in_specs=[
            pl.BlockSpec((NC, blk, D), lambda i: (0, i, 0)),
            pl.BlockSpec((1, D), lambda i: (0, 0)),
        ],
        out_specs=pl.BlockSpec((blk, D), lambda i: (i, 0)),
        out_shape=jax.ShapeDtypeStruct((N, D), _f32),
    )(outp, bias.reshape(1, D))


def kernel(x, edge_index, edge_attr, W_l, b_l, W_r, b_r, W_e, att, bias):
    src = edge_index[0].astype(_i32)
    dst = edge_index[1].astype(_i32)
    eaf = edge_attr.astype(_f32).reshape(E * DE)
    x_l, x_r = _k1(x, W_l, b_l, W_r, b_r)
    logits, tmax = _k2(x_l, x_r, src, dst, eaf, W_e, att)
    z128 = jnp.zeros((STRIPE, D), _f32)
    alpha = _k3a(dst, logits, tmax, z128)
    outp = _k3b(x_l, src, dst, alpha, z128)
    return _k4(outp, bias)
